# Initial kernel scaffold; baseline (speedup 1.0000x reference)
import functools

import jax
import jax.numpy as jnp
from jax import lax
from jax.experimental import pallas as pl
from jax.experimental.pallas import tpu as pltpu
from jax.experimental.pallas import tpu_sc as plsc

N = 10000
E = 320000
D = 128
NB = 8
KH = 3

NC = 2
NS = 16
NW = NC * NS

NPAD = 10240
RPT = NPAD // NS
ECH = 128
EW = 10240
NCHK = EW // ECH
EPAD = NW * EW
CW = 16

_mesh = plsc.VectorSubcoreMesh(core_axis_name="c", subcore_axis_name="s")


@functools.partial(
    pl.kernel,
    out_type=jax.ShapeDtypeStruct((NC, NPAD, CW), jnp.float32),
    mesh=_mesh,
    scratch_types=[
        pltpu.VMEM((NCHK, ECH), jnp.int32),
        pltpu.VMEM((ECH, CW), jnp.float32),
        pltpu.VMEM((RPT, CW), jnp.float32),
        pltpu.VMEM_SHARED((NPAD, CW), jnp.float32),
    ],
)
def _sc_degree(src_hbm, cnt_hbm, srcv, onesv, zerov, acc):
    c = lax.axis_index("c")
    s = lax.axis_index("s")
    wid = c * NS + s

    def fill_ones(i, _):
        onesv[i, :] = jnp.ones((CW,), jnp.float32)
        return 0

    lax.fori_loop(0, ECH, fill_ones, 0)

    def fill_zero(i, _):
        zerov[i, :] = jnp.zeros((CW,), jnp.float32)
        return 0

    lax.fori_loop(0, RPT, fill_zero, 0)

    pltpu.sync_copy(zerov, acc.at[pl.ds(s * RPT, RPT)])
    plsc.subcore_barrier()

    pltpu.sync_copy(src_hbm.at[wid], srcv)

    def step(j, _):
        pltpu.sync_copy(onesv, acc.at[srcv.at[j]], add=True)
        return 0

    lax.fori_loop(0, NCHK, step, 0)
    plsc.subcore_barrier()

    pltpu.sync_copy(acc.at[pl.ds(s * RPT, RPT)],
                    cnt_hbm.at[c, pl.ds(s * RPT, RPT)])


@functools.partial(
    pl.kernel,
    out_type=jax.ShapeDtypeStruct((NC, NPAD, D), jnp.float32),
    mesh=_mesh,
    scratch_types=[
        pltpu.VMEM((NCHK, ECH), jnp.int32),
        pltpu.VMEM((NCHK, ECH), jnp.int32),
        pltpu.VMEM((2, ECH, D), jnp.float32),
        pltpu.VMEM((ECH, D), jnp.float32),
        pltpu.VMEM_SHARED((NPAD, D), jnp.float32),
        pltpu.SemaphoreType.DMA,
    ],
)
def _sc_hop(hs_hbm, src_hbm, dst_hbm, par_hbm, srcv, dstv, rows, zerov, acc, sem):
    c = lax.axis_index("c")
    s = lax.axis_index("s")
    wid = c * NS + s

    def fill_zero(t, _):
        i = t // (D // 16)
        k = t % (D // 16)
        zerov[i, pl.ds(k * 16, 16)] = jnp.zeros((16,), jnp.float32)
        return 0

    lax.fori_loop(0, ECH * (D // 16), fill_zero, 0)

    for r in range(RPT // ECH):
        pltpu.sync_copy(zerov, acc.at[pl.ds(s * RPT + r * ECH, ECH)])
    plsc.subcore_barrier()

    pltpu.sync_copy(src_hbm.at[wid], srcv)
    pltpu.sync_copy(dst_hbm.at[wid], dstv)

    def step(j, _):
        pltpu.async_copy(hs_hbm.at[srcv.at[j]], rows.at[0], sem).wait()
        pltpu.sync_copy(rows.at[0], acc.at[dstv.at[j]], add=True)
        return 0

    lax.fori_loop(0, NCHK, step, 0)
    plsc.subcore_barrier()

    pltpu.sync_copy(acc.at[pl.ds(s * RPT, RPT)],
                    par_hbm.at[c, pl.ds(s * RPT, RPT)])


BLK = 256
_GRID = NPAD // BLK


def _kan_body(first, last, *refs):
    if first:
        (h_ref, cnt_ref, w_ref, b_ref, out_ref, gs_ref) = refs
        g = h_ref[...]
        acc = b_ref[...] + jnp.zeros((BLK, D), jnp.float32)
    elif last:
        (p_ref, w_ref, b_ref, accin_ref, out_ref) = refs
        g = p_ref[0] + p_ref[1]
        acc = accin_ref[...] + b_ref[...]
    else:
        (p_ref, cnt_ref, w_ref, b_ref, accin_ref, out_ref, gs_ref) = refs
        g = p_ref[0] + p_ref[1]
        acc = accin_ref[...] + b_ref[...]

    if not last:
        cnt = cnt_ref[0, :, 0:1] + cnt_ref[1, :, 0:1]
        inv = 1.0 / jnp.maximum(cnt, 1.0)
        gs_ref[...] = g * inv

    mu = jnp.mean(g, axis=1, keepdims=True)
    var = jnp.mean((g - mu) ** 2, axis=1, keepdims=True)
    hn = (g - mu) * lax.rsqrt(var + 1e-5)
    scale = (NB - 1) / 4.0
    for j in range(NB):
        gj = -2.0 + j * (4.0 / (NB - 1))
        r = jnp.exp(-(((hn - gj) * scale) ** 2))
        acc = acc + jnp.dot(r, w_ref[j], preferred_element_type=jnp.float32)
    out_ref[...] = acc


def _make_kan(first, last):
    body = functools.partial(_kan_body, first, last)
    w_spec = pl.BlockSpec((NB, D, D), lambda i: (0, 0, 0))
    b_spec = pl.BlockSpec((1, D), lambda i: (0, 0))
    row_spec = pl.BlockSpec((BLK, D), lambda i: (i, 0))
    pair_spec = pl.BlockSpec((NC, BLK, D), lambda i: (0, i, 0))
    cnt_spec = pl.BlockSpec((NC, BLK, CW), lambda i: (0, i, 0))
    row_shape = jax.ShapeDtypeStruct((NPAD, D), jnp.float32)

    if first:
        in_specs = [row_spec, cnt_spec, w_spec, b_spec]
        out_specs = [row_spec, row_spec]
        out_shape = [row_shape, row_shape]
        aliases = {}
    elif last:
        in_specs = [pair_spec, w_spec, b_spec, row_spec]
        out_specs = row_spec
        out_shape = row_shape
        aliases = {3: 0}
    else:
        in_specs = [pair_spec, cnt_spec, w_spec, b_spec, row_spec]
        out_specs = [row_spec, row_spec]
        out_shape = [row_shape, row_shape]
        aliases = {4: 0}

    return pl.pallas_call(
        body,
        grid=(_GRID,),
        in_specs=in_specs,
        out_specs=out_specs,
        out_shape=out_shape,
        input_output_aliases=aliases,
    )


_kan_first = _make_kan(True, False)
_kan_mid = _make_kan(False, False)
_kan_last = _make_kan(False, True)


def kernel(x, edge_index, kan_W, kan_b, hop_weights):
    src = edge_index[0]
    dst = edge_index[1]
    pad = jnp.full((EPAD - E,), N, jnp.int32)
    src_p = jnp.concatenate([src, pad]).reshape(NW, NCHK, ECH)
    dst_p = jnp.concatenate([dst, pad]).reshape(NW, NCHK, ECH)
    xp = jnp.pad(x, ((0, NPAD - N), (0, 0)))

    w = jax.nn.softmax(hop_weights)
    Wp = kan_W.reshape(KH + 1, D, NB, D).transpose(0, 2, 1, 3)
    Wp = Wp * w[:, None, None, None]
    bp = (kan_b * w[:, None])[:, None, :]

    cnt = _sc_degree(src_p)
    out0, xs = _kan_first(xp, cnt, Wp[0], bp[0])
    p1 = _sc_hop(xs, src_p, dst_p)
    out1, g1s = _kan_mid(p1, cnt, Wp[1], bp[1], out0)
    p2 = _sc_hop(g1s, src_p, dst_p)
    out2, g2s = _kan_mid(p2, cnt, Wp[2], bp[2], out1)
    p3 = _sc_hop(g2s, src_p, dst_p)
    out3 = _kan_last(p3, Wp[3], bp[3], out2)
    return out3[:N]


# speedup vs baseline: 4.1093x; 4.1093x over previous
import functools

import jax
import jax.numpy as jnp
from jax import lax
from jax.experimental import pallas as pl
from jax.experimental.pallas import tpu as pltpu
from jax.experimental.pallas import tpu_sc as plsc

N = 10000
E = 320000
D = 128
NB = 8
KH = 3

NC = 2
NS = 16
NW = NC * NS

NPAD = 10240
RPT = NPAD // NS
ECH = 128
EW = 10240
NCHK = EW // ECH
EPAD = NW * EW
CW = 128

@functools.cache
def _sc_kernels():
    mesh = plsc.VectorSubcoreMesh(
        core_axis_name="c", subcore_axis_name="s", num_cores=NC, num_subcores=NS
    )
    degree = functools.partial(
        pl.kernel,
        out_type=jax.ShapeDtypeStruct((NC, NPAD, CW), jnp.float32),
        mesh=mesh,
        scratch_types=[
            pltpu.VMEM((NCHK, ECH), jnp.int32),
            pltpu.VMEM((ECH, CW), jnp.float32),
            pltpu.VMEM_SHARED((NPAD, CW), jnp.float32),
        ],
    )(_sc_degree_body)
    hop = functools.partial(
        pl.kernel,
        out_type=jax.ShapeDtypeStruct((NC, NPAD, D), jnp.float32),
        mesh=mesh,
        scratch_types=[
            pltpu.VMEM((NCHK, ECH), jnp.int32),
            pltpu.VMEM((NCHK, ECH), jnp.int32),
            pltpu.VMEM((ECH, D), jnp.float32),
            pltpu.VMEM_SHARED((NPAD, D), jnp.float32),
            pltpu.SemaphoreType.DMA,
        ],
    )(_sc_hop_body)
    return degree, hop


def _sc_degree_body(src_hbm, cnt_hbm, srcv, onesv, acc):
    c = lax.axis_index("c")
    s = lax.axis_index("s")
    wid = c * NS + s

    def fill(val, t, _):
        i = t // (CW // 16)
        k = t % (CW // 16)
        onesv[i, pl.ds(k * 16, 16)] = jnp.full((16,), val, jnp.float32)
        return 0

    lax.fori_loop(0, ECH * (CW // 16), functools.partial(fill, 0.0), 0)
    for r in range(RPT // ECH):
        pltpu.sync_copy(onesv, acc.at[pl.ds(s * RPT + r * ECH, ECH)])
    lax.fori_loop(0, ECH * (CW // 16), functools.partial(fill, 1.0), 0)
    plsc.subcore_barrier()

    pltpu.sync_copy(src_hbm.at[wid], srcv)

    def step(j, _):
        pltpu.sync_copy(onesv, acc.at[srcv.at[j]], add=True)
        return 0

    lax.fori_loop(0, NCHK, step, 0)
    plsc.subcore_barrier()

    pltpu.sync_copy(acc.at[pl.ds(s * RPT, RPT)],
                    cnt_hbm.at[c, pl.ds(s * RPT, RPT)])


def _sc_hop_body(hs_hbm, src_hbm, dst_hbm, par_hbm, srcv, dstv, rows, acc, sem):
    c = lax.axis_index("c")
    s = lax.axis_index("s")
    wid = c * NS + s

    def fill_zero(t, _):
        i = t // (D // 16)
        k = t % (D // 16)
        rows[i, pl.ds(k * 16, 16)] = jnp.zeros((16,), jnp.float32)
        return 0

    lax.fori_loop(0, ECH * (D // 16), fill_zero, 0)

    for r in range(RPT // ECH):
        pltpu.sync_copy(rows, acc.at[pl.ds(s * RPT + r * ECH, ECH)])
    plsc.subcore_barrier()

    pltpu.sync_copy(src_hbm.at[wid], srcv)
    pltpu.sync_copy(dst_hbm.at[wid], dstv)

    def step(j, _):
        pltpu.async_copy(hs_hbm.at[srcv.at[j]], rows, sem).wait()
        pltpu.sync_copy(rows, acc.at[dstv.at[j]], add=True)
        return 0

    lax.fori_loop(0, NCHK, step, 0)
    plsc.subcore_barrier()

    pltpu.sync_copy(acc.at[pl.ds(s * RPT, RPT)],
                    par_hbm.at[c, pl.ds(s * RPT, RPT)])


BLK = 256
_GRID = NPAD // BLK


def _kan_body(first, last, *refs):
    if first:
        (h_ref, cnt_ref, w_ref, b_ref, out_ref, gs_ref) = refs
        g = h_ref[...]
        acc = b_ref[...] + jnp.zeros((BLK, D), jnp.float32)
    elif last:
        (p_ref, w_ref, b_ref, accin_ref, out_ref) = refs
        g = p_ref[0] + p_ref[1]
        acc = accin_ref[...] + b_ref[...]
    else:
        (p_ref, cnt_ref, w_ref, b_ref, accin_ref, out_ref, gs_ref) = refs
        g = p_ref[0] + p_ref[1]
        acc = accin_ref[...] + b_ref[...]

    if not last:
        cnt = cnt_ref[0] + cnt_ref[1]
        inv = 1.0 / jnp.maximum(cnt, 1.0)
        gs_ref[...] = g * inv

    mu = jnp.mean(g, axis=1, keepdims=True)
    var = jnp.mean((g - mu) ** 2, axis=1, keepdims=True)
    hn = (g - mu) * lax.rsqrt(var + 1e-5)
    scale = (NB - 1) / 4.0
    for j in range(NB):
        gj = -2.0 + j * (4.0 / (NB - 1))
        r = jnp.exp(-(((hn - gj) * scale) ** 2))
        acc = acc + jnp.dot(r, w_ref[j], preferred_element_type=jnp.float32)
    out_ref[...] = acc


def _make_kan(first, last):
    body = functools.partial(_kan_body, first, last)
    w_spec = pl.BlockSpec((NB, D, D), lambda i: (0, 0, 0))
    b_spec = pl.BlockSpec((1, D), lambda i: (0, 0))
    row_spec = pl.BlockSpec((BLK, D), lambda i: (i, 0))
    pair_spec = pl.BlockSpec((NC, BLK, D), lambda i: (0, i, 0))
    cnt_spec = pl.BlockSpec((NC, BLK, CW), lambda i: (0, i, 0))
    row_shape = jax.ShapeDtypeStruct((NPAD, D), jnp.float32)

    if first:
        in_specs = [row_spec, cnt_spec, w_spec, b_spec]
        out_specs = [row_spec, row_spec]
        out_shape = [row_shape, row_shape]
        aliases = {}
    elif last:
        in_specs = [pair_spec, w_spec, b_spec, row_spec]
        out_specs = row_spec
        out_shape = row_shape
        aliases = {3: 0}
    else:
        in_specs = [pair_spec, cnt_spec, w_spec, b_spec, row_spec]
        out_specs = [row_spec, row_spec]
        out_shape = [row_shape, row_shape]
        aliases = {4: 0}

    return pl.pallas_call(
        body,
        grid=(_GRID,),
        in_specs=in_specs,
        out_specs=out_specs,
        out_shape=out_shape,
        input_output_aliases=aliases,
    )


_kan_first = _make_kan(True, False)
_kan_mid = _make_kan(False, False)
_kan_last = _make_kan(False, True)


def kernel(x, edge_index, kan_W, kan_b, hop_weights):
    src = edge_index[0]
    dst = edge_index[1]
    pad = jnp.full((EPAD - E,), N, jnp.int32)
    src_p = jnp.concatenate([src, pad]).reshape(NW, NCHK, ECH)
    dst_p = jnp.concatenate([dst, pad]).reshape(NW, NCHK, ECH)
    xp = jnp.pad(x, ((0, NPAD - N), (0, 0)))

    w = jax.nn.softmax(hop_weights)
    Wp = kan_W.reshape(KH + 1, D, NB, D).transpose(0, 2, 1, 3)
    Wp = Wp * w[:, None, None, None]
    bp = (kan_b * w[:, None])[:, None, :]

    sc_degree, sc_hop = _sc_kernels()
    cnt = sc_degree(src_p)
    out0, xs = _kan_first(xp, cnt, Wp[0], bp[0])
    p1 = sc_hop(xs, src_p, dst_p)
    out1, g1s = _kan_mid(p1, cnt, Wp[1], bp[1], out0)
    p2 = sc_hop(g1s, src_p, dst_p)
    out2, g2s = _kan_mid(p2, cnt, Wp[2], bp[2], out1)
    p3 = sc_hop(g2s, src_p, dst_p)
    out3 = _kan_last(p3, Wp[3], bp[3], out2)
    return out3[:N]


# speedup vs baseline: 4.4728x; 1.0885x over previous
import functools

import jax
import jax.numpy as jnp
from jax import lax
from jax.experimental import pallas as pl
from jax.experimental.pallas import tpu as pltpu
from jax.experimental.pallas import tpu_sc as plsc

N = 10000
E = 320000
D = 128
NB = 8
KH = 3

NC = 2
NS = 16
NW = NC * NS

NPAD = 10240
RPT = NPAD // NS
ECH = 128
EW = 10240
NCHK = EW // ECH
HCHK = NCHK // 2
EPAD = NW * EW
CW = 128

@functools.cache
def _sc_kernels():
    mesh = plsc.VectorSubcoreMesh(
        core_axis_name="c", subcore_axis_name="s", num_cores=NC, num_subcores=NS
    )
    degree = functools.partial(
        pl.kernel,
        out_type=jax.ShapeDtypeStruct((NC, NPAD, CW), jnp.float32),
        mesh=mesh,
        scratch_types=[
            pltpu.VMEM((NCHK, ECH), jnp.int32),
            pltpu.VMEM((ECH, CW), jnp.float32),
            pltpu.VMEM_SHARED((NPAD, CW), jnp.float32),
        ],
    )(_sc_degree_body)
    hop = functools.partial(
        pl.kernel,
        out_type=jax.ShapeDtypeStruct((NC, NPAD, D), jnp.float32),
        mesh=mesh,
        scratch_types=[
            pltpu.VMEM((HCHK, ECH), jnp.int32),
            pltpu.VMEM((HCHK, ECH), jnp.int32),
            pltpu.VMEM((2, ECH, D), jnp.float32),
            pltpu.VMEM_SHARED((NPAD, D), jnp.float32),
            pltpu.SemaphoreType.DMA,
        ],
    )(_sc_hop_body)
    return degree, hop


def _sc_degree_body(src_hbm, cnt_hbm, srcv, onesv, acc):
    c = lax.axis_index("c")
    s = lax.axis_index("s")
    wid = c * NS + s

    def fill(val, t, _):
        i = t // (CW // 16)
        k = t % (CW // 16)
        onesv[i, pl.ds(k * 16, 16)] = jnp.full((16,), val, jnp.float32)
        return 0

    lax.fori_loop(0, ECH * (CW // 16), functools.partial(fill, 0.0), 0)
    for r in range(RPT // ECH):
        pltpu.sync_copy(onesv, acc.at[pl.ds(s * RPT + r * ECH, ECH)])
    lax.fori_loop(0, ECH * (CW // 16), functools.partial(fill, 1.0), 0)
    plsc.subcore_barrier()

    pltpu.sync_copy(src_hbm.at[wid], srcv)

    def step(j, _):
        pltpu.sync_copy(onesv, acc.at[srcv.at[j]], add=True)
        return 0

    lax.fori_loop(0, NCHK, step, 0)
    plsc.subcore_barrier()

    pltpu.sync_copy(acc.at[pl.ds(s * RPT, RPT)],
                    cnt_hbm.at[c, pl.ds(s * RPT, RPT)])


def _sc_hop_body(hs_hbm, src_hbm, dst_hbm, par_hbm, srcv, dstv, rows, acc, sem):
    c = lax.axis_index("c")
    s = lax.axis_index("s")
    wid = c * NS + s

    def fill_zero(t, _):
        i = t // (D // 16)
        k = t % (D // 16)
        rows[0, i, pl.ds(k * 16, 16)] = jnp.zeros((16,), jnp.float32)
        return 0

    lax.fori_loop(0, ECH * (D // 16), fill_zero, 0)

    for r in range(RPT // ECH):
        pltpu.sync_copy(rows.at[0], acc.at[pl.ds(s * RPT + r * ECH, ECH)])

    pltpu.sync_copy(src_hbm.at[wid, pl.ds(0, HCHK)], srcv)
    pltpu.sync_copy(dst_hbm.at[wid, pl.ds(0, HCHK)], dstv)
    pltpu.async_copy(hs_hbm.at[srcv.at[0]], rows.at[0], sem)
    plsc.subcore_barrier()

    for h in range(2):
        def step(j, _):
            pltpu.make_async_copy(hs_hbm.at[srcv.at[j]],
                                  rows.at[j % 2], sem).wait()

            @pl.when(j < HCHK - 1)
            def _():
                pltpu.async_copy(hs_hbm.at[srcv.at[j + 1]],
                                 rows.at[(j + 1) % 2], sem)

            pltpu.sync_copy(rows.at[j % 2], acc.at[dstv.at[j]], add=True)
            return 0

        lax.fori_loop(0, HCHK, step, 0)
        if h == 0:
            pltpu.sync_copy(src_hbm.at[wid, pl.ds(HCHK, HCHK)], srcv)
            pltpu.sync_copy(dst_hbm.at[wid, pl.ds(HCHK, HCHK)], dstv)
            pltpu.async_copy(hs_hbm.at[srcv.at[0]], rows.at[0], sem)

    plsc.subcore_barrier()

    pltpu.sync_copy(acc.at[pl.ds(s * RPT, RPT)],
                    par_hbm.at[c, pl.ds(s * RPT, RPT)])


BLK = 256
_GRID = NPAD // BLK


def _kan_body(first, last, *refs):
    if first:
        (h_ref, cnt_ref, w_ref, b_ref, out_ref, gs_ref) = refs
        g = h_ref[...]
        acc = b_ref[...] + jnp.zeros((BLK, D), jnp.float32)
    elif last:
        (p_ref, w_ref, b_ref, accin_ref, out_ref) = refs
        g = p_ref[0] + p_ref[1]
        acc = accin_ref[...] + b_ref[...]
    else:
        (p_ref, cnt_ref, w_ref, b_ref, accin_ref, out_ref, gs_ref) = refs
        g = p_ref[0] + p_ref[1]
        acc = accin_ref[...] + b_ref[...]

    if not last:
        cnt = cnt_ref[0] + cnt_ref[1]
        inv = 1.0 / jnp.maximum(cnt, 1.0)
        gs_ref[...] = g * inv

    mu = jnp.mean(g, axis=1, keepdims=True)
    var = jnp.mean((g - mu) ** 2, axis=1, keepdims=True)
    hn = (g - mu) * lax.rsqrt(var + 1e-5)
    scale = (NB - 1) / 4.0
    for j in range(NB):
        gj = -2.0 + j * (4.0 / (NB - 1))
        r = jnp.exp(-(((hn - gj) * scale) ** 2))
        acc = acc + jnp.dot(r, w_ref[j], preferred_element_type=jnp.float32)
    out_ref[...] = acc


def _make_kan(first, last):
    body = functools.partial(_kan_body, first, last)
    w_spec = pl.BlockSpec((NB, D, D), lambda i: (0, 0, 0))
    b_spec = pl.BlockSpec((1, D), lambda i: (0, 0))
    row_spec = pl.BlockSpec((BLK, D), lambda i: (i, 0))
    pair_spec = pl.BlockSpec((NC, BLK, D), lambda i: (0, i, 0))
    cnt_spec = pl.BlockSpec((NC, BLK, CW), lambda i: (0, i, 0))
    row_shape = jax.ShapeDtypeStruct((NPAD, D), jnp.float32)

    if first:
        in_specs = [row_spec, cnt_spec, w_spec, b_spec]
        out_specs = [row_spec, row_spec]
        out_shape = [row_shape, row_shape]
        aliases = {}
    elif last:
        in_specs = [pair_spec, w_spec, b_spec, row_spec]
        out_specs = row_spec
        out_shape = row_shape
        aliases = {3: 0}
    else:
        in_specs = [pair_spec, cnt_spec, w_spec, b_spec, row_spec]
        out_specs = [row_spec, row_spec]
        out_shape = [row_shape, row_shape]
        aliases = {4: 0}

    return pl.pallas_call(
        body,
        grid=(_GRID,),
        in_specs=in_specs,
        out_specs=out_specs,
        out_shape=out_shape,
        input_output_aliases=aliases,
    )


_kan_first = _make_kan(True, False)
_kan_mid = _make_kan(False, False)
_kan_last = _make_kan(False, True)


def kernel(x, edge_index, kan_W, kan_b, hop_weights):
    src = edge_index[0]
    dst = edge_index[1]
    pad = jnp.full((EPAD - E,), N, jnp.int32)
    src_p = jnp.concatenate([src, pad]).reshape(NW, NCHK, ECH)
    dst_p = jnp.concatenate([dst, pad]).reshape(NW, NCHK, ECH)
    xp = jnp.pad(x, ((0, NPAD - N), (0, 0)))

    w = jax.nn.softmax(hop_weights)
    Wp = kan_W.reshape(KH + 1, D, NB, D).transpose(0, 2, 1, 3)
    Wp = Wp * w[:, None, None, None]
    bp = (kan_b * w[:, None])[:, None, :]

    sc_degree, sc_hop = _sc_kernels()
    cnt = sc_degree(src_p)
    out0, xs = _kan_first(xp, cnt, Wp[0], bp[0])
    p1 = sc_hop(xs, src_p, dst_p)
    out1, g1s = _kan_mid(p1, cnt, Wp[1], bp[1], out0)
    p2 = sc_hop(g1s, src_p, dst_p)
    out2, g2s = _kan_mid(p2, cnt, Wp[2], bp[2], out1)
    p3 = sc_hop(g2s, src_p, dst_p)
    out3 = _kan_last(p3, Wp[3], bp[3], out2)
    return out3[:N]


# speedup vs baseline: 4.6316x; 1.0355x over previous
import functools

import jax
import jax.numpy as jnp
from jax import lax
from jax.experimental import pallas as pl
from jax.experimental.pallas import tpu as pltpu
from jax.experimental.pallas import tpu_sc as plsc

N = 10000
E = 320000
D = 128
NB = 8
KH = 3

NC = 2
NS = 16
NW = NC * NS

NPAD = 10240
RPT = NPAD // NS
ECH = 64
EW = 10240
NCHK = EW // ECH
QCHK = NCHK // 4
ZR = 128
NBUF = 4
EPAD = NW * EW
CW = 128

@functools.cache
def _sc_kernels():
    mesh = plsc.VectorSubcoreMesh(
        core_axis_name="c", subcore_axis_name="s", num_cores=NC, num_subcores=NS
    )
    degree = functools.partial(
        pl.kernel,
        out_type=jax.ShapeDtypeStruct((NC, NPAD, CW), jnp.float32),
        mesh=mesh,
        scratch_types=[
            pltpu.VMEM((NCHK, ECH), jnp.int32),
            pltpu.VMEM((ECH, CW), jnp.float32),
            pltpu.VMEM((ZR, CW), jnp.float32),
            pltpu.VMEM_SHARED((NPAD, CW), jnp.float32),
        ],
    )(_sc_degree_body)
    hop = functools.partial(
        pl.kernel,
        out_type=jax.ShapeDtypeStruct((NC, NPAD, D), jnp.float32),
        mesh=mesh,
        scratch_types=[
            pltpu.VMEM((QCHK, ECH), jnp.int32),
            pltpu.VMEM((QCHK, ECH), jnp.int32),
            pltpu.VMEM((NBUF, ECH, D), jnp.float32),
            pltpu.VMEM_SHARED((NPAD, D), jnp.float32),
            pltpu.SemaphoreType.DMA((NBUF,)),
            pltpu.SemaphoreType.DMA((NBUF,)),
        ],
    )(_sc_hop_body)
    return degree, hop


def _sc_degree_body(src_hbm, cnt_hbm, srcv, onesv, zerov, acc):
    c = lax.axis_index("c")
    s = lax.axis_index("s")
    wid = c * NS + s

    def fill_ones(t, _):
        i = t // (CW // 16)
        k = t % (CW // 16)
        onesv[i, pl.ds(k * 16, 16)] = jnp.full((16,), 1.0, jnp.float32)
        return 0

    def fill_zero(t, _):
        i = t // (CW // 16)
        k = t % (CW // 16)
        zerov[i, pl.ds(k * 16, 16)] = jnp.zeros((16,), jnp.float32)
        return 0

    lax.fori_loop(0, ZR * (CW // 16), fill_zero, 0)
    for r in range(RPT // ZR):
        pltpu.sync_copy(zerov, acc.at[pl.ds(s * RPT + r * ZR, ZR)])
    lax.fori_loop(0, ECH * (CW // 16), fill_ones, 0)
    plsc.subcore_barrier()

    pltpu.sync_copy(src_hbm.at[wid], srcv)

    def step(j, _):
        pltpu.sync_copy(onesv, acc.at[srcv.at[j]], add=True)
        return 0

    lax.fori_loop(0, NCHK, step, 0)
    plsc.subcore_barrier()

    pltpu.sync_copy(acc.at[pl.ds(s * RPT, RPT)],
                    cnt_hbm.at[c, pl.ds(s * RPT, RPT)])


def _sc_hop_body(hs_hbm, src_hbm, dst_hbm, par_hbm, srcv, dstv, rows, acc,
                 gsem, ssem):
    c = lax.axis_index("c")
    s = lax.axis_index("s")
    wid = c * NS + s

    def fill_zero(t, _):
        i = t // (D // 16)
        k = t % (D // 16)
        rows[0, i, pl.ds(k * 16, 16)] = jnp.zeros((16,), jnp.float32)
        return 0

    lax.fori_loop(0, ECH * (D // 16), fill_zero, 0)

    for r in range(RPT // ECH):
        pltpu.sync_copy(rows.at[0], acc.at[pl.ds(s * RPT + r * ECH, ECH)])

    for h in range(4):
        pltpu.sync_copy(src_hbm.at[wid, pl.ds(h * QCHK, QCHK)], srcv)
        pltpu.sync_copy(dst_hbm.at[wid, pl.ds(h * QCHK, QCHK)], dstv)
        if h == 0:
            plsc.subcore_barrier()
        for k in range(NBUF - 1):
            pltpu.async_copy(hs_hbm.at[srcv.at[k]], rows.at[k], gsem.at[k])

        def step(j, _):
            b = j % NBUF
            pltpu.make_async_copy(hs_hbm.at[srcv.at[j]], rows.at[b],
                                  gsem.at[b]).wait()
            pltpu.async_copy(rows.at[b], acc.at[dstv.at[j]], ssem.at[b],
                             add=True)

            nj = j + NBUF - 1
            nb = nj % NBUF

            @pl.when(nj < QCHK)
            def _():
                @pl.when(j > 0)
                def _():
                    pltpu.make_async_copy(rows.at[nb], acc.at[dstv.at[j - 1]],
                                          ssem.at[nb]).wait()

                pltpu.async_copy(hs_hbm.at[srcv.at[nj]], rows.at[nb],
                                 gsem.at[nb])

            return 0

        lax.fori_loop(0, QCHK, step, 0)

        for k in range(NBUF):
            j = QCHK - 1 - k
            pltpu.make_async_copy(rows.at[j % NBUF], acc.at[dstv.at[j]],
                                  ssem.at[j % NBUF]).wait()

    plsc.subcore_barrier()

    pltpu.sync_copy(acc.at[pl.ds(s * RPT, RPT)],
                    par_hbm.at[c, pl.ds(s * RPT, RPT)])


BLK = 256
_GRID = NPAD // BLK


def _kan_body(first, last, *refs):
    if first:
        (h_ref, cnt_ref, w_ref, b_ref, out_ref, gs_ref) = refs
        g = h_ref[...]
        acc = b_ref[...] + jnp.zeros((BLK, D), jnp.float32)
    elif last:
        (p_ref, w_ref, b_ref, accin_ref, out_ref) = refs
        g = p_ref[0] + p_ref[1]
        acc = accin_ref[...] + b_ref[...]
    else:
        (p_ref, cnt_ref, w_ref, b_ref, accin_ref, out_ref, gs_ref) = refs
        g = p_ref[0] + p_ref[1]
        acc = accin_ref[...] + b_ref[...]

    if not last:
        cnt = cnt_ref[0] + cnt_ref[1]
        inv = 1.0 / jnp.maximum(cnt, 1.0)
        gs_ref[...] = g * inv

    mu = jnp.mean(g, axis=1, keepdims=True)
    var = jnp.mean((g - mu) ** 2, axis=1, keepdims=True)
    hn = (g - mu) * lax.rsqrt(var + 1e-5)
    scale = (NB - 1) / 4.0
    for j in range(NB):
        gj = -2.0 + j * (4.0 / (NB - 1))
        r = jnp.exp(-(((hn - gj) * scale) ** 2))
        acc = acc + jnp.dot(r, w_ref[j], preferred_element_type=jnp.float32)
    out_ref[...] = acc


def _make_kan(first, last):
    body = functools.partial(_kan_body, first, last)
    w_spec = pl.BlockSpec((NB, D, D), lambda i: (0, 0, 0))
    b_spec = pl.BlockSpec((1, D), lambda i: (0, 0))
    row_spec = pl.BlockSpec((BLK, D), lambda i: (i, 0))
    pair_spec = pl.BlockSpec((NC, BLK, D), lambda i: (0, i, 0))
    cnt_spec = pl.BlockSpec((NC, BLK, CW), lambda i: (0, i, 0))
    row_shape = jax.ShapeDtypeStruct((NPAD, D), jnp.float32)

    if first:
        in_specs = [row_spec, cnt_spec, w_spec, b_spec]
        out_specs = [row_spec, row_spec]
        out_shape = [row_shape, row_shape]
        aliases = {}
    elif last:
        in_specs = [pair_spec, w_spec, b_spec, row_spec]
        out_specs = row_spec
        out_shape = row_shape
        aliases = {3: 0}
    else:
        in_specs = [pair_spec, cnt_spec, w_spec, b_spec, row_spec]
        out_specs = [row_spec, row_spec]
        out_shape = [row_shape, row_shape]
        aliases = {4: 0}

    return pl.pallas_call(
        body,
        grid=(_GRID,),
        in_specs=in_specs,
        out_specs=out_specs,
        out_shape=out_shape,
        input_output_aliases=aliases,
    )


_kan_first = _make_kan(True, False)
_kan_mid = _make_kan(False, False)
_kan_last = _make_kan(False, True)


def kernel(x, edge_index, kan_W, kan_b, hop_weights):
    src = edge_index[0]
    dst = edge_index[1]
    pad = jnp.full((EPAD - E,), N, jnp.int32)
    src_p = jnp.concatenate([src, pad]).reshape(NW, NCHK, ECH)
    dst_p = jnp.concatenate([dst, pad]).reshape(NW, NCHK, ECH)
    xp = jnp.pad(x, ((0, NPAD - N), (0, 0)))

    w = jax.nn.softmax(hop_weights)
    Wp = kan_W.reshape(KH + 1, D, NB, D).transpose(0, 2, 1, 3)
    Wp = Wp * w[:, None, None, None]
    bp = (kan_b * w[:, None])[:, None, :]

    sc_degree, sc_hop = _sc_kernels()
    cnt = sc_degree(src_p)
    out0, xs = _kan_first(xp, cnt, Wp[0], bp[0])
    p1 = sc_hop(xs, src_p, dst_p)
    out1, g1s = _kan_mid(p1, cnt, Wp[1], bp[1], out0)
    p2 = sc_hop(g1s, src_p, dst_p)
    out2, g2s = _kan_mid(p2, cnt, Wp[2], bp[2], out1)
    p3 = sc_hop(g2s, src_p, dst_p)
    out3 = _kan_last(p3, Wp[3], bp[3], out2)
    return out3[:N]
